# initial kernel scaffold (unmeasured)
import jax
import jax.numpy as jnp
from jax import lax
from jax.experimental import pallas as pl
from jax.experimental.pallas import tpu as pltpu


def kernel(
    x,
):
    def body(*refs):
        pass

    out_shape = jax.ShapeDtypeStruct(..., jnp.float32)
    return pl.pallas_call(body, out_shape=out_shape)(...)


# baseline (device time: 377858 ns/iter reference)
import jax
import jax.numpy as jnp
from jax import lax
from jax.experimental import pallas as pl
from jax.experimental.pallas import tpu as pltpu

N_DEV = 8


def kernel(x):
    x = x.reshape(x.shape[1], x.shape[2]).astype(jnp.bfloat16)
    m, n = x.shape
    m_chunk = m // N_DEV

    def body(x_ref, out_ref, sbuf, recv_buf, rs_send, rs_recv, ag_send, ag_recv):
        me = lax.axis_index("i")
        right = (me + 1) % N_DEV

        sbuf[:, :] = x_ref[pl.ds(me * m_chunk, m_chunk), :]
        for s in range(N_DEV - 1):
            rdma = pltpu.make_async_remote_copy(
                src_ref=sbuf,
                dst_ref=recv_buf.at[s],
                send_sem=rs_send.at[s],
                recv_sem=rs_recv.at[s],
                device_id=(right,),
                device_id_type=pl.DeviceIdType.MESH,
            )
            rdma.start()
            rdma.wait()
            nxt = (me - s - 1) % N_DEV
            sbuf[:, :] = recv_buf[s] + x_ref[pl.ds(nxt * m_chunk, m_chunk), :]

        r_me = (me + 1) % N_DEV
        out_ref[pl.ds(r_me * m_chunk, m_chunk), :] = sbuf[:, :]

        for t in range(N_DEV - 1):
            c = (me + 1 - t) % N_DEV
            rdma = pltpu.make_async_remote_copy(
                src_ref=out_ref.at[pl.ds(c * m_chunk, m_chunk)],
                dst_ref=out_ref.at[pl.ds(c * m_chunk, m_chunk)],
                send_sem=ag_send.at[t],
                recv_sem=ag_recv.at[t],
                device_id=(right,),
                device_id_type=pl.DeviceIdType.MESH,
            )
            rdma.start()
            rdma.wait()

    out_shape = jax.ShapeDtypeStruct((m, n), jnp.bfloat16)
    return pl.pallas_call(
        body,
        out_shape=out_shape,
        in_specs=[pl.BlockSpec(memory_space=pltpu.VMEM)],
        out_specs=pl.BlockSpec(memory_space=pltpu.VMEM),
        scratch_shapes=[
            pltpu.VMEM((m_chunk, n), jnp.bfloat16),
            pltpu.VMEM((N_DEV - 1, m_chunk, n), jnp.bfloat16),
            pltpu.SemaphoreType.DMA((N_DEV - 1,)),
            pltpu.SemaphoreType.DMA((N_DEV - 1,)),
            pltpu.SemaphoreType.DMA((N_DEV - 1,)),
            pltpu.SemaphoreType.DMA((N_DEV - 1,)),
        ],
    )(x)


# device time: 226039 ns/iter; 1.6716x vs baseline; 1.6716x over previous
import jax
import jax.numpy as jnp
from jax import lax
from jax.experimental import pallas as pl
from jax.experimental.pallas import tpu as pltpu

N_DEV = 8


def kernel(x):
    x = x.reshape(x.shape[1], x.shape[2]).astype(jnp.bfloat16)
    m, n = x.shape
    mc = m // N_DEV
    nh = n // 2

    def body(x_ref, out_ref, sb_cw, sb_ccw, rv_cw, rv_ccw, sems):
        me = lax.axis_index("i")
        right = (me + 1) % N_DEV
        left = (me - 1) % N_DEV

        cw = pl.ds(0, nh)
        ccw = pl.ds(nh, nh)

        sb_cw[:, :] = x_ref[pl.ds(me * mc, mc), cw]
        sb_ccw[:, :] = x_ref[pl.ds(me * mc, mc), ccw]
        for s in range(N_DEV - 1):
            r_cw = pltpu.make_async_remote_copy(
                src_ref=sb_cw,
                dst_ref=rv_cw.at[s],
                send_sem=sems.at[0, s, 0],
                recv_sem=sems.at[0, s, 1],
                device_id=(right,),
                device_id_type=pl.DeviceIdType.MESH,
            )
            r_ccw = pltpu.make_async_remote_copy(
                src_ref=sb_ccw,
                dst_ref=rv_ccw.at[s],
                send_sem=sems.at[1, s, 0],
                recv_sem=sems.at[1, s, 1],
                device_id=(left,),
                device_id_type=pl.DeviceIdType.MESH,
            )
            r_cw.start()
            r_ccw.start()
            r_cw.wait()
            r_ccw.wait()
            c_cw = (me - s - 1) % N_DEV
            c_ccw = (me + s + 1) % N_DEV
            sb_cw[:, :] = rv_cw[s] + x_ref[pl.ds(c_cw * mc, mc), cw]
            sb_ccw[:, :] = rv_ccw[s] + x_ref[pl.ds(c_ccw * mc, mc), ccw]

        out_ref[pl.ds(((me + 1) % N_DEV) * mc, mc), cw] = sb_cw[:, :]
        out_ref[pl.ds(((me - 1) % N_DEV) * mc, mc), ccw] = sb_ccw[:, :]

        for t in range(N_DEV - 1):
            c_cw = (me + 1 - t) % N_DEV
            c_ccw = (me - 1 + t) % N_DEV
            g_cw = pltpu.make_async_remote_copy(
                src_ref=out_ref.at[pl.ds(c_cw * mc, mc), cw],
                dst_ref=out_ref.at[pl.ds(c_cw * mc, mc), cw],
                send_sem=sems.at[2, t, 0],
                recv_sem=sems.at[2, t, 1],
                device_id=(right,),
                device_id_type=pl.DeviceIdType.MESH,
            )
            g_ccw = pltpu.make_async_remote_copy(
                src_ref=out_ref.at[pl.ds(c_ccw * mc, mc), ccw],
                dst_ref=out_ref.at[pl.ds(c_ccw * mc, mc), ccw],
                send_sem=sems.at[3, t, 0],
                recv_sem=sems.at[3, t, 1],
                device_id=(left,),
                device_id_type=pl.DeviceIdType.MESH,
            )
            g_cw.start()
            g_ccw.start()
            g_cw.wait()
            g_ccw.wait()

    out_shape = jax.ShapeDtypeStruct((m, n), jnp.bfloat16)
    return pl.pallas_call(
        body,
        out_shape=out_shape,
        in_specs=[pl.BlockSpec(memory_space=pltpu.VMEM)],
        out_specs=pl.BlockSpec(memory_space=pltpu.VMEM),
        scratch_shapes=[
            pltpu.VMEM((mc, nh), jnp.bfloat16),
            pltpu.VMEM((mc, nh), jnp.bfloat16),
            pltpu.VMEM((N_DEV - 1, mc, nh), jnp.bfloat16),
            pltpu.VMEM((N_DEV - 1, mc, nh), jnp.bfloat16),
            pltpu.SemaphoreType.DMA((4, N_DEV - 1, 2)),
        ],
    )(x)


# device time: 214554 ns/iter; 1.7611x vs baseline; 1.0535x over previous
import jax
import jax.numpy as jnp
from jax import lax
from jax.experimental import pallas as pl
from jax.experimental.pallas import tpu as pltpu

N_DEV = 8


def kernel(x):
    x = x.reshape(x.shape[1], x.shape[2])
    m, n = x.shape
    mc = m // N_DEV
    nh = n // 2

    def body(x_ref, out_ref, sb_cw, sb_ccw, rv_cw, rv_ccw,
             xs_cw, xs_ccw, sems, load_sems):
        me = lax.axis_index("i")
        right = (me + 1) % N_DEV
        left = (me - 1) % N_DEV

        cw = pl.ds(0, nh)
        ccw = pl.ds(nh, nh)

        def load(c, col, dst, slot):
            cp = pltpu.make_async_copy(
                src_ref=x_ref.at[pl.ds(c * mc, mc), col],
                dst_ref=dst.at[slot],
                sem=load_sems.at[0 if dst is xs_cw else 1, slot],
            )
            cp.start()
            return cp

        l0_cw = load(me, cw, xs_cw, 0)
        l0_ccw = load(me, ccw, xs_ccw, 0)
        l1_cw = load((me - 1) % N_DEV, cw, xs_cw, 1)
        l1_ccw = load((me + 1) % N_DEV, ccw, xs_ccw, 1)

        barrier_sem = pltpu.get_barrier_semaphore()
        for nbr in (left, right):
            pl.semaphore_signal(
                barrier_sem, inc=1,
                device_id=(nbr,), device_id_type=pl.DeviceIdType.MESH,
            )
        pl.semaphore_wait(barrier_sem, 2)

        l0_cw.wait()
        l0_ccw.wait()
        sb_cw[:, :] = xs_cw[0].astype(jnp.bfloat16)
        sb_ccw[:, :] = xs_ccw[0].astype(jnp.bfloat16)
        pending = (l1_cw, l1_ccw)

        for s in range(N_DEV - 1):
            r_cw = pltpu.make_async_remote_copy(
                src_ref=sb_cw,
                dst_ref=rv_cw.at[s],
                send_sem=sems.at[0, s, 0],
                recv_sem=sems.at[0, s, 1],
                device_id=(right,),
                device_id_type=pl.DeviceIdType.MESH,
            )
            r_ccw = pltpu.make_async_remote_copy(
                src_ref=sb_ccw,
                dst_ref=rv_ccw.at[s],
                send_sem=sems.at[1, s, 0],
                recv_sem=sems.at[1, s, 1],
                device_id=(left,),
                device_id_type=pl.DeviceIdType.MESH,
            )
            r_cw.start()
            r_ccw.start()
            if s < N_DEV - 2:
                nl_cw = load((me - s - 2) % N_DEV, cw, xs_cw, s % 2)
                nl_ccw = load((me + s + 2) % N_DEV, ccw, xs_ccw, s % 2)
            r_cw.wait()
            r_ccw.wait()
            pending[0].wait()
            pending[1].wait()
            slot = (s + 1) % 2
            sb_cw[:, :] = rv_cw[s] + xs_cw[slot].astype(jnp.bfloat16)
            sb_ccw[:, :] = rv_ccw[s] + xs_ccw[slot].astype(jnp.bfloat16)
            if s < N_DEV - 2:
                pending = (nl_cw, nl_ccw)

        out_ref[pl.ds(((me + 1) % N_DEV) * mc, mc), cw] = sb_cw[:, :]
        out_ref[pl.ds(((me - 1) % N_DEV) * mc, mc), ccw] = sb_ccw[:, :]

        for t in range(N_DEV - 1):
            c_cw = (me + 1 - t) % N_DEV
            c_ccw = (me - 1 + t) % N_DEV
            g_cw = pltpu.make_async_remote_copy(
                src_ref=out_ref.at[pl.ds(c_cw * mc, mc), cw],
                dst_ref=out_ref.at[pl.ds(c_cw * mc, mc), cw],
                send_sem=sems.at[2, t, 0],
                recv_sem=sems.at[2, t, 1],
                device_id=(right,),
                device_id_type=pl.DeviceIdType.MESH,
            )
            g_ccw = pltpu.make_async_remote_copy(
                src_ref=out_ref.at[pl.ds(c_ccw * mc, mc), ccw],
                dst_ref=out_ref.at[pl.ds(c_ccw * mc, mc), ccw],
                send_sem=sems.at[3, t, 0],
                recv_sem=sems.at[3, t, 1],
                device_id=(left,),
                device_id_type=pl.DeviceIdType.MESH,
            )
            g_cw.start()
            g_ccw.start()
            g_cw.wait()
            g_ccw.wait()

    out_shape = jax.ShapeDtypeStruct((m, n), jnp.bfloat16)
    return pl.pallas_call(
        body,
        out_shape=out_shape,
        in_specs=[pl.BlockSpec(memory_space=pl.ANY)],
        out_specs=pl.BlockSpec(memory_space=pltpu.VMEM),
        scratch_shapes=[
            pltpu.VMEM((mc, nh), jnp.bfloat16),
            pltpu.VMEM((mc, nh), jnp.bfloat16),
            pltpu.VMEM((N_DEV - 1, mc, nh), jnp.bfloat16),
            pltpu.VMEM((N_DEV - 1, mc, nh), jnp.bfloat16),
            pltpu.VMEM((2, mc, nh), jnp.float32),
            pltpu.VMEM((2, mc, nh), jnp.float32),
            pltpu.SemaphoreType.DMA((4, N_DEV - 1, 2)),
            pltpu.SemaphoreType.DMA((2, 2)),
        ],
        compiler_params=pltpu.CompilerParams(
            collective_id=0, vmem_limit_bytes=100 * 1024 * 1024
        ),
    )(x)


# device time: 185447 ns/iter; 2.0376x vs baseline; 1.1570x over previous
import jax
import jax.numpy as jnp
from jax import lax
from jax.experimental import pallas as pl
from jax.experimental.pallas import tpu as pltpu

N_DEV = 8
N_RING = 4
CW = (True, True, False, False)


def kernel(x):
    x = x.reshape(x.shape[1], x.shape[2])
    m, n = x.shape
    mc = m // N_DEV
    nq = n // N_RING

    def body(x_ref, out_ref, sb, rv, xs, sems, load_sems):
        me = lax.axis_index("i")
        right = (me + 1) % N_DEV
        left = (me - 1) % N_DEV

        def col(r):
            return pl.ds(r * nq, nq)

        def tgt(r):
            return right if CW[r] else left

        def add_chunk(r, s):
            return (me - s - 1) % N_DEV if CW[r] else (me + s + 1) % N_DEV

        def ag_chunk(r, t):
            return (me + 1 - t) % N_DEV if CW[r] else (me - 1 + t) % N_DEV

        def load(r, c, slot):
            cp = pltpu.make_async_copy(
                src_ref=x_ref.at[pl.ds(c * mc, mc), col(r)],
                dst_ref=xs.at[r, slot],
                sem=load_sems.at[r, slot],
            )
            cp.start()
            return cp

        def rs_rdma(r, s):
            return pltpu.make_async_remote_copy(
                src_ref=sb.at[r],
                dst_ref=rv.at[r, s],
                send_sem=sems.at[0, r, s, 0],
                recv_sem=sems.at[0, r, s, 1],
                device_id=(tgt(r),),
                device_id_type=pl.DeviceIdType.MESH,
            )

        def ag_rdma(r, t):
            c = ag_chunk(r, t)
            sl = (pl.ds(c * mc, mc), col(r))
            return pltpu.make_async_remote_copy(
                src_ref=out_ref.at[sl],
                dst_ref=out_ref.at[sl],
                send_sem=sems.at[1, r, t, 0],
                recv_sem=sems.at[1, r, t, 1],
                device_id=(tgt(r),),
                device_id_type=pl.DeviceIdType.MESH,
            )

        l0 = [load(r, me, 0) for r in range(N_RING)]
        l1 = [load(r, add_chunk(r, 0), 1) for r in range(N_RING)]

        barrier_sem = pltpu.get_barrier_semaphore()
        for nbr in (left, right):
            pl.semaphore_signal(
                barrier_sem, inc=1,
                device_id=(nbr,), device_id_type=pl.DeviceIdType.MESH,
            )
        pl.semaphore_wait(barrier_sem, 2)

        cur = [None] * N_RING
        for r in range(N_RING):
            l0[r].wait()
            sb[r] = xs[r, 0].astype(jnp.bfloat16)
            cur[r] = rs_rdma(r, 0)
            cur[r].start()
        pending = l1

        ag_cur = [None] * N_RING
        for s in range(N_DEV - 1):
            for r in range(N_RING):
                if s < N_DEV - 2:
                    nl = load(r, add_chunk(r, s + 1), s % 2)
                cur[r].wait()
                pending[r].wait()
                sb[r] = rv[r, s] + xs[r, (s + 1) % 2].astype(jnp.bfloat16)
                if s < N_DEV - 2:
                    cur[r] = rs_rdma(r, s + 1)
                    cur[r].start()
                    pending[r] = nl
                else:
                    out_ref[pl.ds(ag_chunk(r, 0) * mc, mc), col(r)] = sb[r]
                    ag_cur[r] = ag_rdma(r, 0)
                    ag_cur[r].start()

        for t in range(N_DEV - 1):
            for r in range(N_RING):
                ag_cur[r].wait()
                if t < N_DEV - 2:
                    ag_cur[r] = ag_rdma(r, t + 1)
                    ag_cur[r].start()

    out_shape = jax.ShapeDtypeStruct((m, n), jnp.bfloat16)
    return pl.pallas_call(
        body,
        out_shape=out_shape,
        in_specs=[pl.BlockSpec(memory_space=pl.ANY)],
        out_specs=pl.BlockSpec(memory_space=pltpu.VMEM),
        scratch_shapes=[
            pltpu.VMEM((N_RING, mc, nq), jnp.bfloat16),
            pltpu.VMEM((N_RING, N_DEV - 1, mc, nq), jnp.bfloat16),
            pltpu.VMEM((N_RING, 2, mc, nq), jnp.float32),
            pltpu.SemaphoreType.DMA((2, N_RING, N_DEV - 1, 2)),
            pltpu.SemaphoreType.DMA((N_RING, 2)),
        ],
        compiler_params=pltpu.CompilerParams(
            collective_id=0, vmem_limit_bytes=100 * 1024 * 1024
        ),
    )(x)


# device time: 184365 ns/iter; 2.0495x vs baseline; 1.0059x over previous
import jax
import jax.numpy as jnp
from jax import lax
from jax.experimental import pallas as pl
from jax.experimental.pallas import tpu as pltpu

N_DEV = 8
N_RING = 4
CW = (True, True, False, False)


def kernel(x):
    x = x.reshape(x.shape[1], x.shape[2])
    m, n = x.shape
    mc = m // N_DEV
    nq = n // N_RING

    def body(x_ref, out_ref, sb, rv, xs, sems, load_sems):
        me = lax.axis_index("i")
        right = (me + 1) % N_DEV
        left = (me - 1) % N_DEV

        def col(r):
            return pl.ds(r * nq, nq)

        def tgt(r):
            return right if CW[r] else left

        def add_chunk(r, s):
            return (me - s - 1) % N_DEV if CW[r] else (me + s + 1) % N_DEV

        def ag_chunk(r, t):
            return (me + 1 - t) % N_DEV if CW[r] else (me - 1 + t) % N_DEV

        def load(r, c, slot):
            cp = pltpu.make_async_copy(
                src_ref=x_ref.at[pl.ds(c * mc, mc), col(r)],
                dst_ref=xs.at[r, slot],
                sem=load_sems.at[r, slot],
            )
            cp.start()
            return cp

        def rs_rdma(r, s):
            return pltpu.make_async_remote_copy(
                src_ref=sb.at[r],
                dst_ref=rv.at[r, s],
                send_sem=sems.at[0, r, s, 0],
                recv_sem=sems.at[0, r, s, 1],
                device_id=(tgt(r),),
                device_id_type=pl.DeviceIdType.MESH,
            )

        def ag_rdma(r, t, src=None):
            c = ag_chunk(r, t)
            sl = (pl.ds(c * mc, mc), col(r))
            return pltpu.make_async_remote_copy(
                src_ref=out_ref.at[sl] if src is None else src,
                dst_ref=out_ref.at[sl],
                send_sem=sems.at[1, r, t, 0],
                recv_sem=sems.at[1, r, t, 1],
                device_id=(tgt(r),),
                device_id_type=pl.DeviceIdType.MESH,
            )

        l0 = [load(r, me, 0) for r in range(N_RING)]
        l1 = [load(r, add_chunk(r, 0), 1) for r in range(N_RING)]

        barrier_sem = pltpu.get_barrier_semaphore()
        for nbr in (left, right):
            pl.semaphore_signal(
                barrier_sem, inc=1,
                device_id=(nbr,), device_id_type=pl.DeviceIdType.MESH,
            )
        pl.semaphore_wait(barrier_sem, 2)

        ORDER = (0, 2, 1, 3)

        cur = [None] * N_RING
        for r in ORDER:
            l0[r].wait()
            sb[r] = xs[r, 0].astype(jnp.bfloat16)
            cur[r] = rs_rdma(r, 0)
            cur[r].start()
        pending = l1

        ag_cur = [None] * N_RING
        for s in range(N_DEV - 1):
            for r in ORDER:
                if s < N_DEV - 2:
                    nl = load(r, add_chunk(r, s + 1), s % 2)
                cur[r].wait()
                pending[r].wait()
                sb[r] = rv[r, s] + xs[r, (s + 1) % 2].astype(jnp.bfloat16)
                if s < N_DEV - 2:
                    cur[r] = rs_rdma(r, s + 1)
                    cur[r].start()
                    pending[r] = nl
                else:
                    out_ref[pl.ds(ag_chunk(r, 0) * mc, mc), col(r)] = sb[r]
                    ag_cur[r] = ag_rdma(r, 0)
                    ag_cur[r].start()

        for t in range(N_DEV - 1):
            for r in ORDER:
                ag_cur[r].wait()
                if t < N_DEV - 2:
                    ag_cur[r] = ag_rdma(r, t + 1)
                    ag_cur[r].start()

    out_shape = jax.ShapeDtypeStruct((m, n), jnp.bfloat16)
    return pl.pallas_call(
        body,
        out_shape=out_shape,
        in_specs=[pl.BlockSpec(memory_space=pl.ANY)],
        out_specs=pl.BlockSpec(memory_space=pltpu.VMEM),
        scratch_shapes=[
            pltpu.VMEM((N_RING, mc, nq), jnp.bfloat16),
            pltpu.VMEM((N_RING, N_DEV - 1, mc, nq), jnp.bfloat16),
            pltpu.VMEM((N_RING, 2, mc, nq), jnp.float32),
            pltpu.SemaphoreType.DMA((2, N_RING, N_DEV - 1, 2)),
            pltpu.SemaphoreType.DMA((N_RING, 2)),
        ],
        compiler_params=pltpu.CompilerParams(
            collective_id=0, vmem_limit_bytes=100 * 1024 * 1024
        ),
    )(x)


# device time: 179639 ns/iter; 2.1034x vs baseline; 1.0263x over previous
import jax
import jax.numpy as jnp
from jax import lax
from jax.experimental import pallas as pl
from jax.experimental.pallas import tpu as pltpu

N_DEV = 8
N_RING = 4
CW = (True, True, False, False)


def kernel(x):
    x = x.reshape(x.shape[1], x.shape[2])
    m, n = x.shape
    mc = m // N_DEV
    nq = n // N_RING

    def body(x_ref, out_ref, sb, rv, gt, xs, sems, load_sems, out_sems):
        me = lax.axis_index("i")
        right = (me + 1) % N_DEV
        left = (me - 1) % N_DEV

        def col(r):
            return pl.ds(r * nq, nq)

        def tgt(r):
            return right if CW[r] else left

        def add_chunk(r, s):
            return (me - s - 1) % N_DEV if CW[r] else (me + s + 1) % N_DEV

        def ag_chunk(r, t):
            return (me + 1 - t) % N_DEV if CW[r] else (me - 1 + t) % N_DEV

        def load(r, c, slot):
            cp = pltpu.make_async_copy(
                src_ref=x_ref.at[pl.ds(c * mc, mc), col(r)],
                dst_ref=xs.at[r, slot],
                sem=load_sems.at[r, slot],
            )
            cp.start()
            return cp

        def store_out(r, c, k):
            sl = (pl.ds(c * mc, mc), col(r))
            cp = pltpu.make_async_copy(
                src_ref=gt.at[sl], dst_ref=out_ref.at[sl],
                sem=out_sems.at[r, k],
            )
            cp.start()
            return cp

        def rs_rdma(r, s):
            return pltpu.make_async_remote_copy(
                src_ref=sb.at[r],
                dst_ref=rv.at[r, s],
                send_sem=sems.at[0, r, s, 0],
                recv_sem=sems.at[0, r, s, 1],
                device_id=(tgt(r),),
                device_id_type=pl.DeviceIdType.MESH,
            )

        def ag_rdma(r, t):
            c = ag_chunk(r, t)
            sl = (pl.ds(c * mc, mc), col(r))
            return pltpu.make_async_remote_copy(
                src_ref=gt.at[sl],
                dst_ref=gt.at[sl],
                send_sem=sems.at[1, r, t, 0],
                recv_sem=sems.at[1, r, t, 1],
                device_id=(tgt(r),),
                device_id_type=pl.DeviceIdType.MESH,
            )

        ORDER = (0, 2, 1, 3)

        l0 = [load(r, me, 0) for r in range(N_RING)]
        l1 = [load(r, add_chunk(r, 0), 1) for r in range(N_RING)]

        barrier_sem = pltpu.get_barrier_semaphore()
        for nbr in (left, right):
            pl.semaphore_signal(
                barrier_sem, inc=1,
                device_id=(nbr,), device_id_type=pl.DeviceIdType.MESH,
            )
        pl.semaphore_wait(barrier_sem, 2)

        cur = [None] * N_RING
        for r in ORDER:
            l0[r].wait()
            sb[r] = xs[r, 0].astype(jnp.bfloat16)
            cur[r] = rs_rdma(r, 0)
            cur[r].start()
        pending = l1

        ag_cur = [None] * N_RING
        stores = []
        for s in range(N_DEV - 1):
            for r in ORDER:
                if s < N_DEV - 2:
                    nl = load(r, add_chunk(r, s + 1), s % 2)
                cur[r].wait()
                pending[r].wait()
                sb[r] = rv[r, s] + xs[r, (s + 1) % 2].astype(jnp.bfloat16)
                if s < N_DEV - 2:
                    cur[r] = rs_rdma(r, s + 1)
                    cur[r].start()
                    pending[r] = nl
                else:
                    gt[pl.ds(ag_chunk(r, 0) * mc, mc), col(r)] = sb[r]
                    ag_cur[r] = ag_rdma(r, 0)
                    ag_cur[r].start()
                    stores.append(store_out(r, ag_chunk(r, 0), 0))

        for t in range(N_DEV - 1):
            for r in ORDER:
                ag_cur[r].wait()
                if t < N_DEV - 2:
                    ag_cur[r] = ag_rdma(r, t + 1)
                    ag_cur[r].start()
                stores.append(store_out(r, ag_chunk(r, t + 1), t + 1))

        for cp in stores:
            cp.wait()

    out_shape = jax.ShapeDtypeStruct((m, n), jnp.bfloat16)
    return pl.pallas_call(
        body,
        out_shape=out_shape,
        in_specs=[pl.BlockSpec(memory_space=pl.ANY)],
        out_specs=pl.BlockSpec(memory_space=pl.ANY),
        scratch_shapes=[
            pltpu.VMEM((N_RING, mc, nq), jnp.bfloat16),
            pltpu.VMEM((N_RING, N_DEV - 1, mc, nq), jnp.bfloat16),
            pltpu.VMEM((m, n), jnp.bfloat16),
            pltpu.VMEM((N_RING, 2, mc, nq), jnp.float32),
            pltpu.SemaphoreType.DMA((2, N_RING, N_DEV - 1, 2)),
            pltpu.SemaphoreType.DMA((N_RING, 2)),
            pltpu.SemaphoreType.DMA((N_RING, N_DEV)),
        ],
        compiler_params=pltpu.CompilerParams(
            collective_id=0, vmem_limit_bytes=100 * 1024 * 1024
        ),
    )(x)
